# baseline (speedup 1.0000x reference)
import functools

import jax
import jax.numpy as jnp
from jax.experimental import pallas as pl
from jax.experimental.pallas import tpu as pltpu

N = 10000
D = 512
BM1 = 1024
BM = 1024
BK = 2048
N_PAD = 10240


def _support_kernel(x_ref, w_ref, out_ref):
    i = pl.program_id(0)
    row = jax.lax.broadcasted_iota(jnp.int32, x_ref.shape, 0) + i * BM1
    xb = jnp.where(row < N, x_ref[...], 0.0).astype(jnp.bfloat16)
    wb = w_ref[...].astype(jnp.bfloat16)
    out_ref[...] = jnp.dot(
        xb, wb, preferred_element_type=jnp.float32
    ).astype(jnp.bfloat16)


def _spmm_kernel(adj_ref, s_ref, out_ref):
    k = pl.program_id(1)

    @pl.when(k == 0)
    def _():
        out_ref[...] = jnp.zeros_like(out_ref)

    col = jax.lax.broadcasted_iota(jnp.int32, adj_ref.shape, 1) + k * BK
    a = jnp.where(col < N, adj_ref[...], 0.0).astype(jnp.bfloat16)
    out_ref[...] += jnp.dot(a, s_ref[...], preferred_element_type=jnp.float32)


def kernel(x, adj, W):
    support = pl.pallas_call(
        _support_kernel,
        grid=(N_PAD // BM1,),
        in_specs=[
            pl.BlockSpec((BM1, D), lambda i: (i, 0)),
            pl.BlockSpec((D, D), lambda i: (0, 0)),
        ],
        out_specs=pl.BlockSpec((BM1, D), lambda i: (i, 0)),
        out_shape=jax.ShapeDtypeStruct((N_PAD, D), jnp.bfloat16),
        compiler_params=pltpu.CompilerParams(
            dimension_semantics=("parallel",),
        ),
    )(x, W)

    out = pl.pallas_call(
        _spmm_kernel,
        grid=(N_PAD // BM, N_PAD // BK),
        in_specs=[
            pl.BlockSpec((BM, BK), lambda i, k: (i, k)),
            pl.BlockSpec((BK, D), lambda i, k: (k, 0)),
        ],
        out_specs=pl.BlockSpec((BM, D), lambda i, k: (i, 0)),
        out_shape=jax.ShapeDtypeStruct((N_PAD, D), jnp.float32),
        compiler_params=pltpu.CompilerParams(
            dimension_semantics=("parallel", "arbitrary"),
            vmem_limit_bytes=100 * 1024 * 1024,
        ),
    )(adj, support)
    return out[:N]


# speedup vs baseline: 1.1465x; 1.1465x over previous
import functools

import jax
import jax.numpy as jnp
from jax.experimental import pallas as pl
from jax.experimental.pallas import tpu as pltpu

N = 10000
D = 512
BM1 = 2000
BM = 200


def _support_kernel(x_ref, w_ref, out_ref):
    out_ref[...] = jnp.dot(
        x_ref[...], w_ref[...], preferred_element_type=jnp.float32
    )


def _spmm_kernel(adj_ref, s_ref, out_ref):
    out_ref[...] = jnp.dot(
        adj_ref[...], s_ref[...], preferred_element_type=jnp.float32
    )


def kernel(x, adj, W):
    support = pl.pallas_call(
        _support_kernel,
        grid=(N // BM1,),
        in_specs=[
            pl.BlockSpec((BM1, D), lambda i: (i, 0)),
            pl.BlockSpec((D, D), lambda i: (0, 0)),
        ],
        out_specs=pl.BlockSpec((BM1, D), lambda i: (i, 0)),
        out_shape=jax.ShapeDtypeStruct((N, D), jnp.float32),
        compiler_params=pltpu.CompilerParams(
            dimension_semantics=("parallel",),
        ),
    )(x, W)

    out = pl.pallas_call(
        _spmm_kernel,
        grid=(N // BM,),
        in_specs=[
            pl.BlockSpec((BM, N), lambda i: (i, 0)),
            pl.BlockSpec((N, D), lambda i: (0, 0)),
        ],
        out_specs=pl.BlockSpec((BM, D), lambda i: (i, 0)),
        out_shape=jax.ShapeDtypeStruct((N, D), jnp.float32),
        compiler_params=pltpu.CompilerParams(
            dimension_semantics=("arbitrary",),
            vmem_limit_bytes=100 * 1024 * 1024,
        ),
    )(adj, support)
    return out


# speedup vs baseline: 1.2465x; 1.0872x over previous
import functools

import jax
import jax.numpy as jnp
from jax.experimental import pallas as pl
from jax.experimental.pallas import tpu as pltpu

N = 10000
D = 512
BM = 512
N_PAD = 10240


def _gcn_kernel(adj_ref, x_ref, w_ref, out_ref):
    t = jnp.dot(adj_ref[...], x_ref[...], preferred_element_type=jnp.float32)
    out_ref[...] = jnp.dot(t, w_ref[...], preferred_element_type=jnp.float32)


def kernel(x, adj, W):
    out = pl.pallas_call(
        _gcn_kernel,
        grid=(N_PAD // BM,),
        in_specs=[
            pl.BlockSpec((BM, N), lambda i: (i, 0)),
            pl.BlockSpec((N, D), lambda i: (0, 0)),
            pl.BlockSpec((D, D), lambda i: (0, 0)),
        ],
        out_specs=pl.BlockSpec((BM, D), lambda i: (i, 0)),
        out_shape=jax.ShapeDtypeStruct((N_PAD, D), jnp.float32),
        compiler_params=pltpu.CompilerParams(
            dimension_semantics=("arbitrary",),
            vmem_limit_bytes=100 * 1024 * 1024,
        ),
    )(adj, x, W)
    return out[:N]


# speedup vs baseline: 1.2472x; 1.0006x over previous
import functools

import jax
import jax.numpy as jnp
from jax.experimental import pallas as pl
from jax.experimental.pallas import tpu as pltpu

N = 10000
D = 512
BM = 512
N_PAD = 10240


def _gcn_kernel(adj_ref, x_ref, w_ref, out_ref):
    t = jnp.dot(adj_ref[...], x_ref[...], preferred_element_type=jnp.float32)
    out_ref[...] = jnp.dot(t, w_ref[...], preferred_element_type=jnp.float32)


def kernel(x, adj, W):
    out = pl.pallas_call(
        _gcn_kernel,
        grid=(N_PAD // BM,),
        in_specs=[
            pl.BlockSpec((BM, N), lambda i: (i, 0)),
            pl.BlockSpec((N, D), lambda i: (0, 0)),
            pl.BlockSpec((D, D), lambda i: (0, 0)),
        ],
        out_specs=pl.BlockSpec((BM, D), lambda i: (i, 0)),
        out_shape=jax.ShapeDtypeStruct((N_PAD, D), jnp.float32),
        compiler_params=pltpu.CompilerParams(
            dimension_semantics=("parallel",),
            vmem_limit_bytes=100 * 1024 * 1024,
        ),
    )(adj, x, W)
    return out[:N]
